# Initial kernel scaffold; baseline (speedup 1.0000x reference)
import jax
import jax.numpy as jnp
from jax.experimental import pallas as pl


def kernel(inputs, table, W1, b1, W2, b2):
    raise NotImplementedError("write your pallas kernel here")


# speedup vs baseline: 29.4730x; 29.4730x over previous
import functools

import jax
import jax.numpy as jnp
from jax import lax
from jax.experimental import pallas as pl
from jax.experimental.pallas import tpu as pltpu
from jax.experimental.pallas import tpu_sc as plsc

VOCAB = 1000000
EMBED = 32
HIDDEN = 256
SEQ = 1000
BATCH = 1024

N_IDX = BATCH * SEQ
NC, NS = 2, 16
NW = NC * NS
IDX_PER_W = N_IDX // NW
G = 128
NG = IDX_PER_W // G
NBUF = 2

K = EMBED * SEQ
K_BLK = 1280
NK = K // K_BLK


def _sc_gather(idx3, table):
    mesh = plsc.VectorSubcoreMesh(
        core_axis_name="c", subcore_axis_name="s",
        num_cores=NC, num_subcores=NS)

    @functools.partial(
        pl.kernel,
        out_type=jax.ShapeDtypeStruct((N_IDX, EMBED), jnp.float32),
        mesh=mesh,
        scratch_types=[
            pltpu.VMEM((NG, G), jnp.int32),
            pltpu.VMEM((NBUF, G, EMBED), jnp.float32),
            pltpu.SemaphoreType.DMA((NBUF,)),
        ],
        compiler_params=pltpu.CompilerParams(use_tc_tiling_on_sc=False),
    )
    def gather_kernel(idx_hbm, table_hbm, out_hbm, idx_v, rows_v, gsems):
        wid = lax.axis_index("s") * NC + lax.axis_index("c")
        pltpu.sync_copy(idx_hbm.at[wid], idx_v)
        base = wid * IDX_PER_W

        def start(i, slot):
            return pltpu.async_copy(
                table_hbm.at[idx_v.at[i]], rows_v.at[slot], gsems.at[slot])

        start(0, 0)

        def body(i, _):
            slot = lax.rem(i, NBUF)
            nxt = lax.rem(i + 1, NBUF)

            @pl.when(i + 1 < NG)
            def _():
                start(i + 1, nxt)

            pltpu.make_async_copy(
                table_hbm.at[idx_v.at[i]], rows_v.at[slot], gsems.at[slot]
            ).wait()
            pltpu.sync_copy(rows_v.at[slot], out_hbm.at[pl.ds(base + i * G, G)])
            return 0

        lax.fori_loop(0, NG, body, 0, unroll=False)

    return gather_kernel(idx3, table)


def _mlp_body(emb_ref, w1_ref, b1_ref, w2_ref, b2_ref, out_ref, acc_ref):
    k = pl.program_id(0)
    part = lax.dot_general(
        emb_ref[...], w1_ref[...], (((1,), (1,)), ((), ())),
        preferred_element_type=jnp.float32)

    @pl.when(k == 0)
    def _():
        acc_ref[...] = part

    @pl.when(k > 0)
    def _():
        acc_ref[...] += part

    @pl.when(k == NK - 1)
    def _():
        h = jnp.maximum(acc_ref[...] + b1_ref[...], 0.0)
        logits = lax.dot_general(
            h, w2_ref[...], (((1,), (1,)), ((), ())),
            preferred_element_type=jnp.float32) + b2_ref[...]
        m = jnp.max(logits, axis=1, keepdims=True)
        x = logits - m
        lse = jnp.log(jnp.sum(jnp.exp(x), axis=1, keepdims=True))
        out_ref[...] = x - lse


def _tc_mlp(emb2, w1p, b1, w2, b2):
    return pl.pallas_call(
        _mlp_body,
        grid=(NK,),
        in_specs=[
            pl.BlockSpec((BATCH, K_BLK), lambda k: (0, k)),
            pl.BlockSpec((HIDDEN, K_BLK), lambda k: (0, k)),
            pl.BlockSpec((1, HIDDEN), lambda k: (0, 0)),
            pl.BlockSpec((6, HIDDEN), lambda k: (0, 0)),
            pl.BlockSpec((1, 6), lambda k: (0, 0)),
        ],
        out_specs=pl.BlockSpec((BATCH, 6), lambda k: (0, 0)),
        out_shape=jax.ShapeDtypeStruct((BATCH, 6), jnp.float32),
        scratch_shapes=[pltpu.VMEM((BATCH, HIDDEN), jnp.float32)],
    )(emb2, w1p, b1, w2, b2)


def kernel(inputs, table, W1, b1, W2, b2):
    idx3 = inputs.astype(jnp.int32).reshape(NW, NG, G)
    emb = _sc_gather(idx3, table)
    emb2 = emb.reshape(BATCH, K)
    w1p = W1.reshape(HIDDEN, EMBED, SEQ).swapaxes(1, 2).reshape(HIDDEN, K)
    return _tc_mlp(emb2, w1p, b1.reshape(1, HIDDEN), W2, b2.reshape(1, 6))


# speedup vs baseline: 29.9577x; 1.0164x over previous
import functools

import jax
import jax.numpy as jnp
from jax import lax
from jax.experimental import pallas as pl
from jax.experimental.pallas import tpu as pltpu
from jax.experimental.pallas import tpu_sc as plsc

VOCAB = 1000000
EMBED = 32
HIDDEN = 256
SEQ = 1000
BATCH = 1024

N_IDX = BATCH * SEQ
NC, NS = 2, 16
NW = NC * NS
IDX_PER_W = N_IDX // NW
G = 128
NG = IDX_PER_W // G
NBUF = 2

K = EMBED * SEQ
K_BLK = 1280
NK = K // K_BLK


def _sc_gather(idx3, table):
    mesh = plsc.VectorSubcoreMesh(
        core_axis_name="c", subcore_axis_name="s",
        num_cores=NC, num_subcores=NS)

    @functools.partial(
        pl.kernel,
        out_type=jax.ShapeDtypeStruct((N_IDX, EMBED), jnp.float32),
        mesh=mesh,
        scratch_types=[
            pltpu.VMEM((NG, G), jnp.int32),
            pltpu.VMEM((NBUF, G, EMBED), jnp.float32),
            pltpu.SemaphoreType.DMA((NBUF,)),
        ],
        compiler_params=pltpu.CompilerParams(use_tc_tiling_on_sc=False),
    )
    def gather_kernel(idx_hbm, table_hbm, out_hbm, idx_v, rows_v, gsems):
        wid = lax.axis_index("s") * NC + lax.axis_index("c")
        pltpu.sync_copy(idx_hbm.at[wid], idx_v)
        base = wid * IDX_PER_W

        def start(i, slot):
            return pltpu.async_copy(
                table_hbm.at[idx_v.at[i]], rows_v.at[slot], gsems.at[slot])

        start(0, 0)

        def body(i, _):
            slot = lax.rem(i, NBUF)
            nxt = lax.rem(i + 1, NBUF)

            @pl.when(i + 1 < NG)
            def _():
                start(i + 1, nxt)

            pltpu.make_async_copy(
                table_hbm.at[idx_v.at[i]], rows_v.at[slot], gsems.at[slot]
            ).wait()
            pltpu.sync_copy(rows_v.at[slot], out_hbm.at[pl.ds(base + i * G, G)])
            return 0

        lax.fori_loop(0, NG, body, 0, unroll=False)

    return gather_kernel(idx3, table)


def _w1_perm_body(w_ref, out_ref):
    w = w_ref[...]
    out_ref[...] = jnp.transpose(w, (0, 2, 1)).reshape(32, K)


def _w1_permute(W1):
    w13 = W1.reshape(HIDDEN, EMBED, SEQ)
    return pl.pallas_call(
        _w1_perm_body,
        grid=(HIDDEN // 32,),
        in_specs=[pl.BlockSpec((32, EMBED, SEQ), lambda k: (k, 0, 0))],
        out_specs=pl.BlockSpec((32, K), lambda k: (k, 0)),
        out_shape=jax.ShapeDtypeStruct((HIDDEN, K), jnp.float32),
    )(w13)


def _mlp_body(emb_ref, w1_ref, b1_ref, w2_ref, b2_ref, out_ref, acc_ref):
    k = pl.program_id(0)
    part = lax.dot_general(
        emb_ref[...], w1_ref[...], (((1,), (1,)), ((), ())),
        preferred_element_type=jnp.float32)

    @pl.when(k == 0)
    def _():
        acc_ref[...] = part

    @pl.when(k > 0)
    def _():
        acc_ref[...] += part

    @pl.when(k == NK - 1)
    def _():
        h = jnp.maximum(acc_ref[...] + b1_ref[...], 0.0)
        logits = lax.dot_general(
            h, w2_ref[...], (((1,), (1,)), ((), ())),
            preferred_element_type=jnp.float32) + b2_ref[...]
        m = jnp.max(logits, axis=1, keepdims=True)
        x = logits - m
        lse = jnp.log(jnp.sum(jnp.exp(x), axis=1, keepdims=True))
        out_ref[...] = x - lse


def _tc_mlp(emb2, w1p, b1, w2, b2):
    return pl.pallas_call(
        _mlp_body,
        grid=(NK,),
        in_specs=[
            pl.BlockSpec((BATCH, K_BLK), lambda k: (0, k)),
            pl.BlockSpec((HIDDEN, K_BLK), lambda k: (0, k)),
            pl.BlockSpec((1, HIDDEN), lambda k: (0, 0)),
            pl.BlockSpec((6, HIDDEN), lambda k: (0, 0)),
            pl.BlockSpec((1, 6), lambda k: (0, 0)),
        ],
        out_specs=pl.BlockSpec((BATCH, 6), lambda k: (0, 0)),
        out_shape=jax.ShapeDtypeStruct((BATCH, 6), jnp.float32),
        scratch_shapes=[pltpu.VMEM((BATCH, HIDDEN), jnp.float32)],
    )(emb2, w1p, b1, w2, b2)


def kernel(inputs, table, W1, b1, W2, b2):
    idx3 = inputs.astype(jnp.int32).reshape(NW, NG, G)
    emb = _sc_gather(idx3, table)
    emb2 = emb.reshape(BATCH, K)
    w1p = _w1_permute(W1)
    return _tc_mlp(emb2, w1p, b1.reshape(1, HIDDEN), W2, b2.reshape(1, 6))
